# Initial kernel scaffold; baseline (speedup 1.0000x reference)
import jax
import jax.numpy as jnp
from jax.experimental import pallas as pl


def kernel(input_ids, embedding_table):
    raise NotImplementedError("write your pallas kernel here")


# speedup vs baseline: 3.5463x; 3.5463x over previous
import functools

import jax
import jax.numpy as jnp
from jax import lax
from jax.experimental import pallas as pl
from jax.experimental.pallas import tpu as pltpu
from jax.experimental.pallas import tpu_sc as plsc

POSITION_SIZE = 8192
EMBEDDING_SIZE = 128
BATCH = 4
SEQ_LEN = 8192

NUM_CORES = 2
NUM_SUBCORES = 16
NUM_WORKERS = NUM_CORES * NUM_SUBCORES
TOTAL_IDS = BATCH * SEQ_LEN
IDS_PER_WORKER = TOTAL_IDS // NUM_WORKERS
CHUNK = 128
CHUNKS_PER_WORKER = IDS_PER_WORKER // CHUNK
NBUF = 4

_MESH = plsc.VectorSubcoreMesh(core_axis_name="c", subcore_axis_name="s")


@functools.partial(
    pl.kernel,
    mesh=_MESH,
    out_type=jax.ShapeDtypeStruct((TOTAL_IDS, EMBEDDING_SIZE), jnp.float32),
    scratch_types=[
        pltpu.VMEM((CHUNKS_PER_WORKER, CHUNK), jnp.int32),
        pltpu.VMEM((NBUF, CHUNK, EMBEDDING_SIZE), jnp.float32),
        pltpu.SemaphoreType.DMA,
        pltpu.SemaphoreType.DMA,
    ],
)
def _gather_kernel(idx_hbm, table_hbm, out_hbm, idx_v, rows_v, gsem, osem):
    wid = lax.axis_index("s") * NUM_CORES + lax.axis_index("c")
    row_base = wid * IDS_PER_WORKER

    pltpu.sync_copy(idx_hbm.at[pl.ds(wid * CHUNKS_PER_WORKER, CHUNKS_PER_WORKER)],
                    idx_v)

    gathers = [None] * CHUNKS_PER_WORKER
    outs = [None] * CHUNKS_PER_WORKER
    for c in range(min(NBUF, CHUNKS_PER_WORKER)):
        gathers[c] = pltpu.async_copy(
            table_hbm.at[idx_v.at[c]], rows_v.at[c % NBUF], gsem)
    for c in range(CHUNKS_PER_WORKER):
        gathers[c].wait()
        outs[c] = pltpu.async_copy(
            rows_v.at[c % NBUF],
            out_hbm.at[pl.ds(row_base + c * CHUNK, CHUNK)],
            osem)
        nxt = c + NBUF
        if nxt < CHUNKS_PER_WORKER:
            outs[c].wait()
            gathers[nxt] = pltpu.async_copy(
                table_hbm.at[idx_v.at[nxt]], rows_v.at[nxt % NBUF], gsem)
    for c in range(max(0, CHUNKS_PER_WORKER - NBUF), CHUNKS_PER_WORKER):
        outs[c].wait()


def kernel(input_ids, embedding_table):
    idx2d = input_ids.reshape(TOTAL_IDS // CHUNK, CHUNK)
    out = _gather_kernel(idx2d, embedding_table)
    return out.reshape(BATCH, SEQ_LEN, EMBEDDING_SIZE), embedding_table
